# Initial kernel scaffold; baseline (speedup 1.0000x reference)
import functools
import math

import jax
import jax.numpy as jnp
from jax import lax
from jax.experimental import pallas as pl
from jax.experimental.pallas import tpu as pltpu
from jax.experimental.pallas import tpu_sc as plsc

N = 10000
E = 320000
D_IN = 128
H = 128
K = 600
TLD = H * 3 + 1
NC, NS = 2, 16
NW = NC * NS
CH = 128
NCHUNK = E // CH
NROWCH = N // CH
ROWTAIL = N - NROWCH * CH
GP = 768
GPW = GP // NW
SW = 512
TW = 16384
PADV = 2.0


def _vsmesh():
    return plsc.VectorSubcoreMesh(
        core_axis_name="c", subcore_axis_name="s", num_cores=NC, num_subcores=NS
    )


def _fill_1d(ref, n, val):
    for j in range(n // 16):
        ref[pl.ds(j * 16, 16)] = jnp.full((16,), val, jnp.float32)


def _zero_rows(ref, nrow, ncol):
    def body(i, carry):
        for j in range(ncol // 16):
            ref[i, pl.ds(j * 16, 16)] = jnp.zeros((16,), jnp.float32)
        return carry

    lax.fori_loop(0, nrow, body, 0)


def _sc_degrees_body(src_hbm, dst_hbm, out_hbm, idx_v, ones_v, buf_v, dego_sh, degi_sh):
    c = lax.axis_index("c")
    s = lax.axis_index("s")
    wid = s * NC + c
    _fill_1d(ones_v, CH, 1.0)
    _fill_1d(buf_v, CH, 0.0)

    nz = (NROWCH - s + NS - 1) // NS

    def zbody(t, carry):
        off = pl.multiple_of((s + t * NS) * CH, CH)
        pltpu.sync_copy(buf_v, dego_sh.at[pl.ds(off, CH)])
        pltpu.sync_copy(buf_v, degi_sh.at[pl.ds(off, CH)])
        return carry

    lax.fori_loop(0, nz, zbody, 0)

    @pl.when(s == 0)
    def _():
        pltpu.sync_copy(buf_v.at[pl.ds(0, ROWTAIL)], dego_sh.at[pl.ds(NROWCH * CH, ROWTAIL)])
        pltpu.sync_copy(buf_v.at[pl.ds(0, ROWTAIL)], degi_sh.at[pl.ds(NROWCH * CH, ROWTAIL)])

    plsc.subcore_barrier()

    nt = (NCHUNK - wid + NW - 1) // NW

    def ebody(t, carry):
        off = pl.multiple_of((wid + t * NW) * CH, CH)
        pltpu.sync_copy(src_hbm.at[pl.ds(off, CH)], idx_v)
        pltpu.sync_copy(ones_v, dego_sh.at[idx_v], add=True)
        pltpu.sync_copy(dst_hbm.at[pl.ds(off, CH)], idx_v)
        pltpu.sync_copy(ones_v, degi_sh.at[idx_v], add=True)
        return carry

    lax.fori_loop(0, nt, ebody, 0)
    plsc.subcore_barrier()

    def xbody(t, carry):
        off = pl.multiple_of((s + t * NS) * CH, CH)
        pltpu.sync_copy(dego_sh.at[pl.ds(off, CH)], buf_v)
        pltpu.sync_copy(buf_v, out_hbm.at[c, 0, pl.ds(off, CH)])
        pltpu.sync_copy(degi_sh.at[pl.ds(off, CH)], buf_v)
        pltpu.sync_copy(buf_v, out_hbm.at[c, 1, pl.ds(off, CH)])
        return carry

    lax.fori_loop(0, nz, xbody, 0)

    @pl.when(s == 0)
    def _():
        tail = pl.ds(NROWCH * CH, ROWTAIL)
        pltpu.sync_copy(dego_sh.at[tail], buf_v.at[pl.ds(0, ROWTAIL)])
        pltpu.sync_copy(buf_v.at[pl.ds(0, ROWTAIL)], out_hbm.at[c, 0, tail])
        pltpu.sync_copy(degi_sh.at[tail], buf_v.at[pl.ds(0, ROWTAIL)])
        pltpu.sync_copy(buf_v.at[pl.ds(0, ROWTAIL)], out_hbm.at[c, 1, tail])


_sc_degrees = pl.kernel(
    _sc_degrees_body,
    out_type=jax.ShapeDtypeStruct((NC, 2, N), jnp.float32),
    mesh=_vsmesh(),
    scratch_types=[
        pltpu.VMEM((CH,), jnp.int32),
        pltpu.VMEM((CH,), jnp.float32),
        pltpu.VMEM((CH,), jnp.float32),
        pltpu.VMEM_SHARED((N,), jnp.float32),
        pltpu.VMEM_SHARED((N,), jnp.float32),
    ],
)


def _sc_rowscatter_body(h_hbm, src_hbm, dst_hbm, out_hbm, idxs_v, idxd_v, rows_v, sem, agg_sh):
    c = lax.axis_index("c")
    s = lax.axis_index("s")
    wid = s * NC + c
    _zero_rows(rows_v, CH, H)

    nz = (NROWCH - s + NS - 1) // NS

    def zbody(t, carry):
        off = pl.multiple_of((s + t * NS) * CH, CH)
        pltpu.sync_copy(rows_v, agg_sh.at[pl.ds(off, CH)])
        return carry

    lax.fori_loop(0, nz, zbody, 0)

    @pl.when(s == 0)
    def _():
        pltpu.sync_copy(rows_v.at[pl.ds(0, ROWTAIL)], agg_sh.at[pl.ds(NROWCH * CH, ROWTAIL)])

    plsc.subcore_barrier()

    nt = (NCHUNK - wid + NW - 1) // NW

    def ebody(t, carry):
        off = pl.multiple_of((wid + t * NW) * CH, CH)
        pltpu.sync_copy(src_hbm.at[pl.ds(off, CH)], idxs_v)
        pltpu.sync_copy(dst_hbm.at[pl.ds(off, CH)], idxd_v)
        pltpu.async_copy(h_hbm.at[idxs_v], rows_v, sem).wait()
        pltpu.sync_copy(rows_v, agg_sh.at[idxd_v], add=True)
        return carry

    lax.fori_loop(0, nt, ebody, 0)
    plsc.subcore_barrier()

    def xbody(t, carry):
        off = pl.multiple_of((s + t * NS) * CH, CH)
        pltpu.sync_copy(agg_sh.at[pl.ds(off, CH)], rows_v)
        pltpu.sync_copy(rows_v, out_hbm.at[c, pl.ds(off, CH)])
        return carry

    lax.fori_loop(0, nz, xbody, 0)

    @pl.when(s == 0)
    def _():
        tail = pl.ds(NROWCH * CH, ROWTAIL)
        pltpu.sync_copy(agg_sh.at[tail], rows_v.at[pl.ds(0, ROWTAIL)])
        pltpu.sync_copy(rows_v.at[pl.ds(0, ROWTAIL)], out_hbm.at[c, tail])


_sc_rowscatter = pl.kernel(
    _sc_rowscatter_body,
    out_type=jax.ShapeDtypeStruct((NC, N, H), jnp.float32),
    mesh=_vsmesh(),
    scratch_types=[
        pltpu.VMEM((CH,), jnp.int32),
        pltpu.VMEM((CH,), jnp.int32),
        pltpu.VMEM((CH, H), jnp.float32),
        pltpu.SemaphoreType.DMA,
        pltpu.VMEM_SHARED((N, H), jnp.float32),
    ],
)


def _sc_scalarscatter_body(h4_hbm, src_hbm, dst_hbm, out_hbm, h4_v, idxs_v, idxd_v, vals_v, buf_v, acc_sh):
    c = lax.axis_index("c")
    s = lax.axis_index("s")
    wid = s * NC + c
    pltpu.sync_copy(h4_hbm, h4_v)
    _fill_1d(buf_v, CH, 0.0)

    nz = (NROWCH - s + NS - 1) // NS

    def zbody(t, carry):
        off = pl.multiple_of((s + t * NS) * CH, CH)
        pltpu.sync_copy(buf_v, acc_sh.at[pl.ds(off, CH)])
        return carry

    lax.fori_loop(0, nz, zbody, 0)

    @pl.when(s == 0)
    def _():
        pltpu.sync_copy(buf_v.at[pl.ds(0, ROWTAIL)], acc_sh.at[pl.ds(NROWCH * CH, ROWTAIL)])

    plsc.subcore_barrier()

    nt = (NCHUNK - wid + NW - 1) // NW

    def ebody(t, carry):
        off = pl.multiple_of((wid + t * NW) * CH, CH)
        pltpu.sync_copy(src_hbm.at[pl.ds(off, CH)], idxs_v)
        pltpu.sync_copy(dst_hbm.at[pl.ds(off, CH)], idxd_v)
        for j in range(CH // 16):
            iv = idxs_v[pl.ds(j * 16, 16)]
            vals_v[pl.ds(j * 16, 16)] = plsc.load_gather(h4_v, [iv])
        pltpu.sync_copy(vals_v, acc_sh.at[idxd_v], add=True)
        return carry

    lax.fori_loop(0, nt, ebody, 0)
    plsc.subcore_barrier()

    def xbody(t, carry):
        off = pl.multiple_of((s + t * NS) * CH, CH)
        pltpu.sync_copy(acc_sh.at[pl.ds(off, CH)], buf_v)
        pltpu.sync_copy(buf_v, out_hbm.at[c, pl.ds(off, CH)])
        return carry

    lax.fori_loop(0, nz, xbody, 0)

    @pl.when(s == 0)
    def _():
        tail = pl.ds(NROWCH * CH, ROWTAIL)
        pltpu.sync_copy(acc_sh.at[tail], buf_v.at[pl.ds(0, ROWTAIL)])
        pltpu.sync_copy(buf_v.at[pl.ds(0, ROWTAIL)], out_hbm.at[c, tail])


_sc_scalarscatter = pl.kernel(
    _sc_scalarscatter_body,
    out_type=jax.ShapeDtypeStruct((NC, N), jnp.float32),
    mesh=_vsmesh(),
    scratch_types=[
        pltpu.VMEM((N,), jnp.float32),
        pltpu.VMEM((CH,), jnp.int32),
        pltpu.VMEM((CH,), jnp.int32),
        pltpu.VMEM((CH,), jnp.float32),
        pltpu.VMEM((CH,), jnp.float32),
        pltpu.VMEM_SHARED((N,), jnp.float32),
    ],
)


def _sc_gather_body(tab_hbm, idx_hbm, out_hbm, idx_v, rows_v, sem):
    c = lax.axis_index("c")
    s = lax.axis_index("s")
    wid = s * NC + c
    off = pl.multiple_of(wid * GPW, 8)
    pltpu.sync_copy(idx_hbm.at[pl.ds(off, GPW)], idx_v)
    pltpu.async_copy(tab_hbm.at[idx_v], rows_v, sem).wait()
    pltpu.sync_copy(rows_v, out_hbm.at[pl.ds(off, GPW)])


_sc_gather = pl.kernel(
    _sc_gather_body,
    out_type=jax.ShapeDtypeStruct((GP, SW), jnp.float32),
    mesh=_vsmesh(),
    scratch_types=[
        pltpu.VMEM((GPW,), jnp.int32),
        pltpu.VMEM((GPW, SW), jnp.float32),
        pltpu.SemaphoreType.DMA,
    ],
)


def _tc_prep_body(degp_ref, x_ref, w1_ref, h1_ref, ns_ref, nd_ref):
    degp = degp_ref[...]
    deg_o = 1.0 + degp[0, 0] + degp[1, 0]
    deg_i = 1.0 + degp[0, 1] + degp[1, 1]
    ns = lax.rsqrt(deg_o)
    nd = lax.rsqrt(deg_i)
    ns_ref[...] = ns
    nd_ref[...] = nd
    h1_ref[...] = jnp.dot(x_ref[...] * ns[:, None], w1_ref[...])


def _tc_prep(degp, x, w1):
    return pl.pallas_call(
        _tc_prep_body,
        out_shape=[
            jax.ShapeDtypeStruct((N, H), jnp.float32),
            jax.ShapeDtypeStruct((N,), jnp.float32),
            jax.ShapeDtypeStruct((N,), jnp.float32),
        ],
    )(degp, x, w1)


def _tc_layer_body(p_ref, h_ref, nd_ref, b_ref, wn_ref, ns_ref, rm_ref, g_ref, hn_ref, rmax_ref):
    p = p_ref[...]
    agg = p[0] + p[1] + h_ref[...]
    g = jnp.tanh(agg * nd_ref[...][:, None] + b_ref[...][None, :])
    g_ref[...] = g
    hn_ref[...] = jnp.dot(g * ns_ref[...][:, None], wn_ref[...])
    rmax_ref[...] = jnp.maximum(rm_ref[...], jnp.max(g, axis=1))


def _tc_layer(p, h, nd, b, wn, ns, rm):
    hn = wn.shape[1]
    return pl.pallas_call(
        _tc_layer_body,
        out_shape=[
            jax.ShapeDtypeStruct((N, H), jnp.float32),
            jax.ShapeDtypeStruct((N, hn), jnp.float32),
            jax.ShapeDtypeStruct((N,), jnp.float32),
        ],
    )(p, h, nd, b, wn, ns, rm)


def _tc_final_body(p_ref, h4_ref, nd_ref, b4_ref, rm_ref, g4_ref, rmax_ref):
    p = p_ref[...]
    agg = p[0] + p[1] + h4_ref[...]
    g4 = jnp.tanh(agg * nd_ref[...] + b4_ref[...])
    g4_ref[...] = g4
    rmax_ref[...] = jnp.maximum(rm_ref[...], g4)


def _tc_final(p4, h4, nd, b4, rm):
    return pl.pallas_call(
        _tc_final_body,
        out_shape=[
            jax.ShapeDtypeStruct((N,), jnp.float32),
            jax.ShapeDtypeStruct((N,), jnp.float32),
        ],
    )(p4, h4, nd, b4, rm)


def _partner(x, j, cols):
    if j < cols:
        lo = jnp.roll(x, -j, axis=1)
        hi = jnp.roll(x, j, axis=1)
    else:
        lo = jnp.roll(x, -(j // cols), axis=0)
        hi = jnp.roll(x, j // cols, axis=0)
    return lo, hi


def _tc_topk_body(keys_ref, idx_ref):
    R, C = TW // 128, 128
    kv = keys_ref[...]
    pos = lax.broadcasted_iota(jnp.int32, (R, C), 0) * C + lax.broadcasted_iota(
        jnp.int32, (R, C), 1
    )
    iv = pos
    kk = 2
    while kk <= TW:
        j = kk // 2
        while j >= 1:
            klo, khi = _partner(kv, j, C)
            ilo, ihi = _partner(iv, j, C)
            bit = (pos & j) == 0
            pk = jnp.where(bit, klo, khi)
            pi = jnp.where(bit, ilo, ihi)
            less = (kv > pk) | ((kv == pk) & (iv < pi))
            asc = (pos & kk) == 0
            keep = less == (bit == asc)
            kv = jnp.where(keep, kv, pk)
            iv = jnp.where(keep, iv, pi)
            j //= 2
        kk *= 2
    idx_ref[...] = iv


def _tc_topk(keys2d):
    return pl.pallas_call(
        _tc_topk_body,
        out_shape=jax.ShapeDtypeStruct((TW // 128, 128), jnp.int32),
    )(keys2d)


def _tc_head_body(rows_ref, wc1_ref, bc1_ref, apool_ref, wc2_ref, bc2_ref, out_ref):
    x = rows_ref[...]
    pos = lax.broadcasted_iota(jnp.int32, (K, SW), 1)
    kk = 2
    while kk <= SW:
        j = kk // 2
        while j >= 1:
            lo = jnp.roll(x, -j, axis=1)
            hi = jnp.roll(x, j, axis=1)
            bit = (pos & j) == 0
            px = jnp.where(bit, lo, hi)
            less = x < px
            asc = (pos & kk) == 0
            keep = less == (bit == asc)
            x = jnp.where(keep, x, px)
            j //= 2
        kk *= 2
    x = jnp.where(pos < TLD, x, 0.0)
    z = jnp.maximum(jnp.dot(x, wc1_ref[...]) + bc1_ref[...][None, :], 0.0)
    pooled = jnp.dot(apool_ref[...], z)
    t = wc2_ref[...] * pooled[None, :, :]
    out = jnp.maximum(jnp.sum(jnp.sum(t, axis=2), axis=1) + bc2_ref[...], 0.0)
    out_ref[...] = out[None, :]


def _tc_head(rows, wc1p, bc1, apool, wc2t, bc2):
    return pl.pallas_call(
        _tc_head_body,
        out_shape=jax.ShapeDtypeStruct((1, 32), jnp.float32),
    )(rows, wc1p, bc1, apool, wc2t, bc2)


def kernel(X, edge_index, W1, b1, W2, b2, W3, b3, W4, b4, Wc1, bc1, Wc2, bc2):
    src = edge_index[0]
    dst = edge_index[1]

    degp = _sc_degrees(src, dst)
    h1, ns, nd = _tc_prep(degp, X, W1)

    rm0 = jnp.full((N,), -jnp.inf, jnp.float32)
    p1 = _sc_rowscatter(h1, src, dst)
    g1, h2, rm1 = _tc_layer(p1, h1, nd, b1, W2, ns, rm0)
    p2 = _sc_rowscatter(h2, src, dst)
    g2, h3, rm2 = _tc_layer(p2, h2, nd, b2, W3, ns, rm1)
    p3 = _sc_rowscatter(h3, src, dst)
    g3, h4_2d, rm3 = _tc_layer(p3, h3, nd, b3, W4, ns, rm2)

    h4 = h4_2d.reshape(N)
    p4 = _sc_scalarscatter(h4, src, dst)
    g4, rmax = _tc_final(p4, h4, nd, b4, rm3)

    keys2d = jnp.pad(rmax, (0, TW - N), constant_values=-jnp.inf).reshape(TW // 128, 128)
    idx2d = _tc_topk(keys2d)
    idx768 = jnp.pad(idx2d.reshape(-1)[:K], (0, GP - K))

    gcat = jnp.concatenate(
        [g1, g2, g3, g4[:, None], jnp.full((N, SW - TLD), PADV, jnp.float32)], axis=1
    )
    rows = _sc_gather(gcat, idx768)

    wc1p = jnp.pad(jnp.transpose(Wc1[:, 0, :]), ((0, SW - TLD), (0, 0)))
    apool = 0.5 * (
        (jnp.arange(K)[None, :] // 2) == jnp.arange(K // 2)[:, None]
    ).astype(jnp.float32)
    wc2t = jnp.transpose(Wc2, (0, 2, 1))

    return _tc_head(rows[:K], wc1p, bc1, apool, wc2t, bc2)


# speedup vs baseline: 8.8812x; 8.8812x over previous
import functools

import jax
import jax.numpy as jnp
from jax import lax
from jax.experimental import pallas as pl
from jax.experimental.pallas import tpu as pltpu
from jax.experimental.pallas import tpu_sc as plsc

N = 10000
NP = 10240
E = 320000
D_IN = 128
H = 128
K = 600
TLD = H * 3 + 1
NC, NS = 2, 16
NW = NC * NS
CH = 128
NCHUNK = E // CH
NROWCH = NP // CH
NZ = NROWCH // NS
GP = 768
GPW = 128
SW = 512
TW = 16384
PADV = 2.0


def _vsmesh():
    return plsc.VectorSubcoreMesh(
        core_axis_name="c", subcore_axis_name="s", num_cores=NC, num_subcores=NS
    )


def _fill_1d(ref, n, val):
    for j in range(n // 16):
        ref[pl.ds(j * 16, 16)] = jnp.full((16,), val, jnp.float32)


def _zero_rows(ref, nrow, ncol):
    def body(i, carry):
        for j in range(ncol // 16):
            ref[i, pl.ds(j * 16, 16)] = jnp.zeros((16,), jnp.float32)
        return carry

    lax.fori_loop(0, nrow, body, 0)


def _sc_degrees_body(src_hbm, dst_hbm, out_hbm, idx_v, ones_v, buf_v, dego_sh, degi_sh):
    c = lax.axis_index("c")
    s = lax.axis_index("s")
    wid = s * NC + c
    _fill_1d(ones_v, CH, 1.0)
    _fill_1d(buf_v, CH, 0.0)

    def zbody(t, carry):
        off = pl.multiple_of((s + t * NS) * CH, CH)
        pltpu.sync_copy(buf_v, dego_sh.at[pl.ds(off, CH)])
        pltpu.sync_copy(buf_v, degi_sh.at[pl.ds(off, CH)])
        return carry

    lax.fori_loop(0, NZ, zbody, 0)
    plsc.subcore_barrier()

    nt = (NCHUNK - wid + NW - 1) // NW

    def ebody(t, carry):
        off = pl.multiple_of((wid + t * NW) * CH, CH)
        pltpu.sync_copy(src_hbm.at[pl.ds(off, CH)], idx_v)
        pltpu.sync_copy(ones_v, dego_sh.at[idx_v], add=True)
        pltpu.sync_copy(dst_hbm.at[pl.ds(off, CH)], idx_v)
        pltpu.sync_copy(ones_v, degi_sh.at[idx_v], add=True)
        return carry

    lax.fori_loop(0, nt, ebody, 0)
    plsc.subcore_barrier()

    def xbody(t, carry):
        off = pl.multiple_of((s + t * NS) * CH, CH)
        pltpu.sync_copy(dego_sh.at[pl.ds(off, CH)], buf_v)
        pltpu.sync_copy(buf_v, out_hbm.at[c, 0, pl.ds(off, CH)])
        pltpu.sync_copy(degi_sh.at[pl.ds(off, CH)], buf_v)
        pltpu.sync_copy(buf_v, out_hbm.at[c, 1, pl.ds(off, CH)])
        return carry

    lax.fori_loop(0, NZ, xbody, 0)


@functools.cache
def _sc_degrees_call():
    return pl.kernel(
        _sc_degrees_body,
        out_type=jax.ShapeDtypeStruct((NC, 2, NP), jnp.float32),
        mesh=_vsmesh(),
        scratch_types=[
            pltpu.VMEM((CH,), jnp.int32),
            pltpu.VMEM((CH,), jnp.float32),
            pltpu.VMEM((CH,), jnp.float32),
            pltpu.VMEM_SHARED((NP,), jnp.float32),
            pltpu.VMEM_SHARED((NP,), jnp.float32),
        ],
    )


def _sc_degrees(src, dst):
    return _sc_degrees_call()(src, dst)


def _sc_rowscatter_body(h_hbm, src_hbm, dst_hbm, out_hbm, idxs_v, idxd_v, rows_v, sem, agg_sh):
    c = lax.axis_index("c")
    s = lax.axis_index("s")
    wid = s * NC + c
    _zero_rows(rows_v, CH, H)

    def zbody(t, carry):
        off = pl.multiple_of((s + t * NS) * CH, CH)
        pltpu.sync_copy(rows_v, agg_sh.at[pl.ds(off, CH)])
        return carry

    lax.fori_loop(0, NZ, zbody, 0)
    plsc.subcore_barrier()

    nt = (NCHUNK - wid + NW - 1) // NW

    def ebody(t, carry):
        off = pl.multiple_of((wid + t * NW) * CH, CH)
        pltpu.sync_copy(src_hbm.at[pl.ds(off, CH)], idxs_v)
        pltpu.sync_copy(dst_hbm.at[pl.ds(off, CH)], idxd_v)
        pltpu.async_copy(h_hbm.at[idxs_v], rows_v, sem).wait()
        pltpu.sync_copy(rows_v, agg_sh.at[idxd_v], add=True)
        return carry

    lax.fori_loop(0, nt, ebody, 0)
    plsc.subcore_barrier()

    def xbody(t, carry):
        off = pl.multiple_of((s + t * NS) * CH, CH)
        pltpu.sync_copy(agg_sh.at[pl.ds(off, CH)], rows_v)
        pltpu.sync_copy(rows_v, out_hbm.at[c, pl.ds(off, CH)])
        return carry

    lax.fori_loop(0, NZ, xbody, 0)


@functools.cache
def _sc_rowscatter_call():
    return pl.kernel(
        _sc_rowscatter_body,
        out_type=jax.ShapeDtypeStruct((NC, NP, H), jnp.float32),
        mesh=_vsmesh(),
        scratch_types=[
            pltpu.VMEM((CH,), jnp.int32),
            pltpu.VMEM((CH,), jnp.int32),
            pltpu.VMEM((CH, H), jnp.float32),
            pltpu.SemaphoreType.DMA,
            pltpu.VMEM_SHARED((NP, H), jnp.float32),
        ],
    )


def _sc_rowscatter(h, src, dst):
    return _sc_rowscatter_call()(h, src, dst)


def _sc_scalarscatter_body(h4_hbm, src_hbm, dst_hbm, out_hbm, idxs_v, idxd_v, vals_v, buf_v, sem, acc_sh):
    c = lax.axis_index("c")
    s = lax.axis_index("s")
    wid = s * NC + c
    _fill_1d(buf_v, CH, 0.0)

    def zbody(t, carry):
        off = pl.multiple_of((s + t * NS) * CH, CH)
        pltpu.sync_copy(buf_v, acc_sh.at[pl.ds(off, CH)])
        return carry

    lax.fori_loop(0, NZ, zbody, 0)
    plsc.subcore_barrier()

    nt = (NCHUNK - wid + NW - 1) // NW

    def ebody(t, carry):
        off = pl.multiple_of((wid + t * NW) * CH, CH)
        pltpu.sync_copy(src_hbm.at[pl.ds(off, CH)], idxs_v)
        pltpu.sync_copy(dst_hbm.at[pl.ds(off, CH)], idxd_v)
        pltpu.async_copy(h4_hbm.at[idxs_v], vals_v, sem).wait()
        pltpu.sync_copy(vals_v, acc_sh.at[idxd_v], add=True)
        return carry

    lax.fori_loop(0, nt, ebody, 0)
    plsc.subcore_barrier()

    def xbody(t, carry):
        off = pl.multiple_of((s + t * NS) * CH, CH)
        pltpu.sync_copy(acc_sh.at[pl.ds(off, CH)], buf_v)
        pltpu.sync_copy(buf_v, out_hbm.at[c, pl.ds(off, CH)])
        return carry

    lax.fori_loop(0, NZ, xbody, 0)


@functools.cache
def _sc_scalarscatter_call():
    return pl.kernel(
        _sc_scalarscatter_body,
        out_type=jax.ShapeDtypeStruct((NC, NP), jnp.float32),
        mesh=_vsmesh(),
        scratch_types=[
            pltpu.VMEM((CH,), jnp.int32),
            pltpu.VMEM((CH,), jnp.int32),
            pltpu.VMEM((CH,), jnp.float32),
            pltpu.VMEM((CH,), jnp.float32),
            pltpu.SemaphoreType.DMA,
            pltpu.VMEM_SHARED((NP,), jnp.float32),
        ],
    )


def _sc_scalarscatter(h4, src, dst):
    return _sc_scalarscatter_call()(h4, src, dst)


def _sc_gather_body(tab_hbm, idx_hbm, out_hbm, idx_v, rows_v, sem):
    c = lax.axis_index("c")
    s = lax.axis_index("s")
    wid = s * NC + c

    @pl.when(wid < GP // GPW)
    def _():
        off = pl.multiple_of(wid * GPW, CH)
        pltpu.sync_copy(idx_hbm.at[pl.ds(off, GPW)], idx_v)
        pltpu.async_copy(tab_hbm.at[idx_v], rows_v, sem).wait()
        pltpu.sync_copy(rows_v, out_hbm.at[pl.ds(off, GPW)])


@functools.cache
def _sc_gather_call():
    return pl.kernel(
        _sc_gather_body,
        out_type=jax.ShapeDtypeStruct((GP, SW), jnp.float32),
        mesh=_vsmesh(),
        scratch_types=[
            pltpu.VMEM((GPW,), jnp.int32),
            pltpu.VMEM((GPW, SW), jnp.float32),
            pltpu.SemaphoreType.DMA,
        ],
    )


def _sc_gather(tab, idx):
    return _sc_gather_call()(tab, idx)


def _tc_prep_body(degp_ref, x_ref, w1_ref, h1_ref, ns_ref, nd_ref):
    degp = degp_ref[...]
    deg_o = 1.0 + degp[0, 0, :N] + degp[1, 0, :N]
    deg_i = 1.0 + degp[0, 1, :N] + degp[1, 1, :N]
    ns = lax.rsqrt(deg_o)
    nd = lax.rsqrt(deg_i)
    ns_ref[...] = ns
    nd_ref[...] = nd
    h1_ref[...] = jnp.dot(x_ref[...] * ns[:, None], w1_ref[...])


def _tc_prep(degp, x, w1):
    return pl.pallas_call(
        _tc_prep_body,
        out_shape=[
            jax.ShapeDtypeStruct((N, H), jnp.float32),
            jax.ShapeDtypeStruct((N,), jnp.float32),
            jax.ShapeDtypeStruct((N,), jnp.float32),
        ],
    )(degp, x, w1)


def _tc_layer_body(p_ref, h_ref, nd_ref, b_ref, wn_ref, ns_ref, rm_ref, g_ref, hn_ref, rmax_ref):
    p = p_ref[...]
    agg = p[0, :N] + p[1, :N] + h_ref[...]
    g = jnp.tanh(agg * nd_ref[...][:, None] + b_ref[...][None, :])
    g_ref[...] = g
    hn_ref[...] = jnp.dot(g * ns_ref[...][:, None], wn_ref[...])
    rmax_ref[...] = jnp.maximum(rm_ref[...], jnp.max(g, axis=1))


def _tc_layer(p, h, nd, b, wn, ns, rm):
    hn = wn.shape[1]
    return pl.pallas_call(
        _tc_layer_body,
        out_shape=[
            jax.ShapeDtypeStruct((N, H), jnp.float32),
            jax.ShapeDtypeStruct((N, hn), jnp.float32),
            jax.ShapeDtypeStruct((N,), jnp.float32),
        ],
    )(p, h, nd, b, wn, ns, rm)


def _tc_final_body(p_ref, h4_ref, nd_ref, b4_ref, rm_ref, g4_ref, rmax_ref):
    p = p_ref[...]
    agg = p[0, :N] + p[1, :N] + h4_ref[...]
    g4 = jnp.tanh(agg * nd_ref[...] + b4_ref[...])
    g4_ref[...] = g4
    rmax_ref[...] = jnp.maximum(rm_ref[...], g4)


def _tc_final(p4, h4, nd, b4, rm):
    return pl.pallas_call(
        _tc_final_body,
        out_shape=[
            jax.ShapeDtypeStruct((N,), jnp.float32),
            jax.ShapeDtypeStruct((N,), jnp.float32),
        ],
    )(p4, h4, nd, b4, rm)


def _partner(x, j, cols):
    if j < cols:
        lo = jnp.roll(x, -j, axis=1)
        hi = jnp.roll(x, j, axis=1)
    else:
        lo = jnp.roll(x, -(j // cols), axis=0)
        hi = jnp.roll(x, j // cols, axis=0)
    return lo, hi


def _tc_topk_body(keys_ref, idx_ref):
    R, C = TW // 128, 128
    kv = keys_ref[...]
    pos = lax.broadcasted_iota(jnp.int32, (R, C), 0) * C + lax.broadcasted_iota(
        jnp.int32, (R, C), 1
    )
    iv = pos
    kk = 2
    while kk <= TW:
        j = kk // 2
        while j >= 1:
            klo, khi = _partner(kv, j, C)
            ilo, ihi = _partner(iv, j, C)
            bit = (pos & j) == 0
            pk = jnp.where(bit, klo, khi)
            pi = jnp.where(bit, ilo, ihi)
            less = (kv > pk) | ((kv == pk) & (iv < pi))
            asc = (pos & kk) == 0
            keep = less == (bit == asc)
            kv = jnp.where(keep, kv, pk)
            iv = jnp.where(keep, iv, pi)
            j //= 2
        kk *= 2
    idx_ref[...] = iv


def _tc_topk(keys2d):
    return pl.pallas_call(
        _tc_topk_body,
        out_shape=jax.ShapeDtypeStruct((TW // 128, 128), jnp.int32),
    )(keys2d)


def _tc_head_body(rows_ref, wc1_ref, bc1_ref, apool_ref, wc2_ref, bc2_ref, out_ref):
    x = rows_ref[...]
    pos = lax.broadcasted_iota(jnp.int32, (K, SW), 1)
    kk = 2
    while kk <= SW:
        j = kk // 2
        while j >= 1:
            lo = jnp.roll(x, -j, axis=1)
            hi = jnp.roll(x, j, axis=1)
            bit = (pos & j) == 0
            px = jnp.where(bit, lo, hi)
            less = x < px
            asc = (pos & kk) == 0
            keep = less == (bit == asc)
            x = jnp.where(keep, x, px)
            j //= 2
        kk *= 2
    x = jnp.where(pos < TLD, x, 0.0)
    z = jnp.maximum(jnp.dot(x, wc1_ref[...]) + bc1_ref[...][None, :], 0.0)
    pooled = jnp.dot(apool_ref[...], z)
    t = wc2_ref[...] * pooled[None, :, :]
    out = jnp.maximum(jnp.sum(jnp.sum(t, axis=2), axis=1) + bc2_ref[...], 0.0)
    out_ref[...] = out[None, :]


def _tc_head(rows, wc1p, bc1, apool, wc2t, bc2):
    return pl.pallas_call(
        _tc_head_body,
        out_shape=jax.ShapeDtypeStruct((1, 32), jnp.float32),
    )(rows, wc1p, bc1, apool, wc2t, bc2)


def kernel(X, edge_index, W1, b1, W2, b2, W3, b3, W4, b4, Wc1, bc1, Wc2, bc2):
    src = edge_index[0]
    dst = edge_index[1]

    degp = _sc_degrees(src, dst)
    h1, ns, nd = _tc_prep(degp, X, W1)

    rm0 = jnp.full((N,), -jnp.inf, jnp.float32)
    p1 = _sc_rowscatter(h1, src, dst)
    g1, h2, rm1 = _tc_layer(p1, h1, nd, b1, W2, ns, rm0)
    p2 = _sc_rowscatter(h2, src, dst)
    g2, h3, rm2 = _tc_layer(p2, h2, nd, b2, W3, ns, rm1)
    p3 = _sc_rowscatter(h3, src, dst)
    g3, h4_2d, rm3 = _tc_layer(p3, h3, nd, b3, W4, ns, rm2)

    h4 = h4_2d.reshape(N)
    h4p = jnp.pad(h4, (0, NP - N))
    p4 = _sc_scalarscatter(h4p, src, dst)
    g4, rmax = _tc_final(p4, h4, nd, b4, rm3)

    keys2d = jnp.pad(rmax, (0, TW - N), constant_values=-jnp.inf).reshape(TW // 128, 128)
    idx2d = _tc_topk(keys2d)
    idx768 = jnp.pad(idx2d.reshape(-1)[:K], (0, GP - K))

    gcat = jnp.concatenate(
        [g1, g2, g3, g4[:, None], jnp.full((N, SW - TLD), PADV, jnp.float32)], axis=1
    )
    rows = _sc_gather(gcat, idx768)

    wc1p = jnp.pad(jnp.transpose(Wc1[:, 0, :]), ((0, SW - TLD), (0, 0)))
    apool = 0.5 * (
        (jnp.arange(K)[None, :] // 2) == jnp.arange(K // 2)[:, None]
    ).astype(jnp.float32)
    wc2t = jnp.transpose(Wc2, (0, 2, 1))

    return _tc_head(rows[:K], wc1p, bc1, apool, wc2t, bc2)


# speedup vs baseline: 15.1269x; 1.7032x over previous
import functools

import jax
import jax.numpy as jnp
from jax import lax
from jax.experimental import pallas as pl
from jax.experimental.pallas import tpu as pltpu
from jax.experimental.pallas import tpu_sc as plsc

N = 10000
NP = 10240
E = 320000
D_IN = 128
H = 128
K = 600
TLD = H * 3 + 1
NC, NS = 2, 16
NW = NC * NS
CH = 128
NCHUNK = E // CH
NROWCH = NP // CH
NZ = NROWCH // NS
GP = 768
GPW = 128
SW = 512
TW = 16384
PADV = 2.0


def _vsmesh():
    return plsc.VectorSubcoreMesh(
        core_axis_name="c", subcore_axis_name="s", num_cores=NC, num_subcores=NS
    )


def _fill_1d(ref, n, val):
    for j in range(n // 16):
        ref[pl.ds(j * 16, 16)] = jnp.full((16,), val, jnp.float32)


def _zero_rows(ref, nrow, ncol):
    def body(i, carry):
        for j in range(ncol // 16):
            ref[i, pl.ds(j * 16, 16)] = jnp.zeros((16,), jnp.float32)
        return carry

    lax.fori_loop(0, nrow, body, 0)


ECHUNKS = 2560
EPAD = ECHUNKS * CH
CPW = ECHUNKS // NW
SUB = 40
NB = CPW // SUB


def _sc_degrees_body(src2_hbm, dst2_hbm, out_hbm, idxs_v, idxd_v, ones_v, buf_v,
                     semg, sems, dego_sh, degi_sh):
    c = lax.axis_index("c")
    s = lax.axis_index("s")
    wid = s * NC + c
    _fill_1d(ones_v, CH, 1.0)
    _fill_1d(buf_v, CH, 0.0)

    def zbody(t, carry):
        off = pl.multiple_of((s * NZ + t) * CH, CH)
        pltpu.sync_copy(buf_v, dego_sh.at[pl.ds(off, CH)])
        pltpu.sync_copy(buf_v, degi_sh.at[pl.ds(off, CH)])
        return carry

    lax.fori_loop(0, NZ, zbody, 0)
    plsc.subcore_barrier()

    base = wid * CPW

    def batch(boff, nsub):
        pltpu.sync_copy(src2_hbm.at[pl.ds(boff, nsub)], idxs_v.at[pl.ds(0, nsub)])
        pltpu.sync_copy(dst2_hbm.at[pl.ds(boff, nsub)], idxd_v.at[pl.ds(0, nsub)])
        ds = [None] * nsub
        dd = [None] * nsub
        for j in range(nsub):
            if j >= 2:
                ds[j - 2].wait()
                dd[j - 2].wait()
            ds[j] = pltpu.async_copy(ones_v, dego_sh.at[idxs_v.at[j]], semg, add=True)
            dd[j] = pltpu.async_copy(ones_v, degi_sh.at[idxd_v.at[j]], sems, add=True)
        for j in range(max(0, nsub - 2), nsub):
            ds[j].wait()
            dd[j].wait()

    def bloop(b, carry):
        batch(base + b * SUB, SUB)
        return carry

    lax.fori_loop(0, NB, bloop, 0)
    plsc.subcore_barrier()

    def xbody(t, carry):
        off = pl.multiple_of((s * NZ + t) * CH, CH)
        pltpu.sync_copy(dego_sh.at[pl.ds(off, CH)], buf_v)
        pltpu.sync_copy(buf_v, out_hbm.at[c, 0, pl.ds(off, CH)])
        pltpu.sync_copy(degi_sh.at[pl.ds(off, CH)], buf_v)
        pltpu.sync_copy(buf_v, out_hbm.at[c, 1, pl.ds(off, CH)])
        return carry

    lax.fori_loop(0, NZ, xbody, 0)


@functools.cache
def _sc_degrees_call():
    return pl.kernel(
        _sc_degrees_body,
        out_type=jax.ShapeDtypeStruct((NC, 2, NP), jnp.float32),
        mesh=_vsmesh(),
        scratch_types=[
            pltpu.VMEM((SUB, CH), jnp.int32),
            pltpu.VMEM((SUB, CH), jnp.int32),
            pltpu.VMEM((CH,), jnp.float32),
            pltpu.VMEM((CH,), jnp.float32),
            pltpu.SemaphoreType.DMA,
            pltpu.SemaphoreType.DMA,
            pltpu.VMEM_SHARED((NP,), jnp.float32),
            pltpu.VMEM_SHARED((NP,), jnp.float32),
        ],
    )


def _sc_degrees(src2, dst2):
    return _sc_degrees_call()(src2, dst2)


def _sc_rowscatter_body(h_hbm, src2_hbm, dst2_hbm, out_hbm,
                        idxs_v, idxd_v, rows0_v, rows1_v,
                        semg0, semg1, agg_sh):
    c = lax.axis_index("c")
    s = lax.axis_index("s")
    wid = s * NC + c
    _zero_rows(rows0_v, CH, H)

    def zbody(t, carry):
        off = pl.multiple_of((s * NZ + t) * CH, CH)
        pltpu.sync_copy(rows0_v, agg_sh.at[pl.ds(off, CH)])
        return carry

    lax.fori_loop(0, NZ, zbody, 0)
    plsc.subcore_barrier()

    base = wid * CPW
    rbufs = (rows0_v, rows1_v)
    gsems = (semg0, semg1)

    def batch(boff, nsub):
        pltpu.sync_copy(src2_hbm.at[pl.ds(boff, nsub)], idxs_v.at[pl.ds(0, nsub)])
        pltpu.sync_copy(dst2_hbm.at[pl.ds(boff, nsub)], idxd_v.at[pl.ds(0, nsub)])
        gd = [None] * nsub
        gd[0] = pltpu.async_copy(h_hbm.at[idxs_v.at[0]], rbufs[0], gsems[0])
        for j in range(nsub):
            gd[j].wait()
            if j + 1 < nsub:
                gd[j + 1] = pltpu.async_copy(
                    h_hbm.at[idxs_v.at[j + 1]], rbufs[(j + 1) % 2], gsems[(j + 1) % 2]
                )
            pltpu.sync_copy(rbufs[j % 2], agg_sh.at[idxd_v.at[j]], add=True)

    def bloop(b, carry):
        batch(base + b * SUB, SUB)
        return carry

    lax.fori_loop(0, NB, bloop, 0)
    plsc.subcore_barrier()

    def xbody(t, carry):
        off = pl.multiple_of((s * NZ + t) * CH, CH)
        pltpu.sync_copy(agg_sh.at[pl.ds(off, CH)], rows0_v)
        pltpu.sync_copy(rows0_v, out_hbm.at[c, pl.ds(off, CH)])
        return carry

    lax.fori_loop(0, NZ, xbody, 0)


@functools.cache
def _sc_rowscatter_call():
    return pl.kernel(
        _sc_rowscatter_body,
        out_type=jax.ShapeDtypeStruct((NC, NP, H), jnp.float32),
        mesh=_vsmesh(),
        scratch_types=[
            pltpu.VMEM((SUB, CH), jnp.int32),
            pltpu.VMEM((SUB, CH), jnp.int32),
            pltpu.VMEM((CH, H), jnp.float32),
            pltpu.VMEM((CH, H), jnp.float32),
            pltpu.SemaphoreType.DMA,
            pltpu.SemaphoreType.DMA,
            pltpu.VMEM_SHARED((NP, H), jnp.float32),
        ],
    )


def _sc_rowscatter(h, src2, dst2):
    return _sc_rowscatter_call()(h, src2, dst2)


def _sc_scalarscatter_body(h4_hbm, src2_hbm, dst2_hbm, out_hbm,
                           idxs_v, idxd_v, vals0_v, vals1_v, buf_v,
                           semg0, semg1, acc_sh):
    c = lax.axis_index("c")
    s = lax.axis_index("s")
    wid = s * NC + c
    _fill_1d(buf_v, CH, 0.0)

    def zbody(t, carry):
        off = pl.multiple_of((s * NZ + t) * CH, CH)
        pltpu.sync_copy(buf_v, acc_sh.at[pl.ds(off, CH)])
        return carry

    lax.fori_loop(0, NZ, zbody, 0)
    plsc.subcore_barrier()

    base = wid * CPW
    vbufs = (vals0_v, vals1_v)
    gsems = (semg0, semg1)

    def batch(boff, nsub):
        pltpu.sync_copy(src2_hbm.at[pl.ds(boff, nsub)], idxs_v.at[pl.ds(0, nsub)])
        pltpu.sync_copy(dst2_hbm.at[pl.ds(boff, nsub)], idxd_v.at[pl.ds(0, nsub)])
        gd = [None] * nsub
        gd[0] = pltpu.async_copy(h4_hbm.at[idxs_v.at[0]], vbufs[0], gsems[0])
        for j in range(nsub):
            gd[j].wait()
            if j + 1 < nsub:
                gd[j + 1] = pltpu.async_copy(
                    h4_hbm.at[idxs_v.at[j + 1]], vbufs[(j + 1) % 2], gsems[(j + 1) % 2]
                )
            pltpu.sync_copy(vbufs[j % 2], acc_sh.at[idxd_v.at[j]], add=True)

    def bloop(b, carry):
        batch(base + b * SUB, SUB)
        return carry

    lax.fori_loop(0, NB, bloop, 0)
    plsc.subcore_barrier()

    def xbody(t, carry):
        off = pl.multiple_of((s * NZ + t) * CH, CH)
        pltpu.sync_copy(acc_sh.at[pl.ds(off, CH)], buf_v)
        pltpu.sync_copy(buf_v, out_hbm.at[c, pl.ds(off, CH)])
        return carry

    lax.fori_loop(0, NZ, xbody, 0)


@functools.cache
def _sc_scalarscatter_call():
    return pl.kernel(
        _sc_scalarscatter_body,
        out_type=jax.ShapeDtypeStruct((NC, NP), jnp.float32),
        mesh=_vsmesh(),
        scratch_types=[
            pltpu.VMEM((SUB, CH), jnp.int32),
            pltpu.VMEM((SUB, CH), jnp.int32),
            pltpu.VMEM((CH,), jnp.float32),
            pltpu.VMEM((CH,), jnp.float32),
            pltpu.VMEM((CH,), jnp.float32),
            pltpu.SemaphoreType.DMA,
            pltpu.SemaphoreType.DMA,
            pltpu.VMEM_SHARED((NP,), jnp.float32),
        ],
    )


def _sc_scalarscatter(h4, src2, dst2):
    return _sc_scalarscatter_call()(h4, src2, dst2)


def _sc_gather_body(tab_hbm, idx_hbm, out_hbm, idx_v, rows_v, sem):
    c = lax.axis_index("c")
    s = lax.axis_index("s")
    wid = s * NC + c

    @pl.when(wid < GP // GPW)
    def _():
        off = pl.multiple_of(wid * GPW, CH)
        pltpu.sync_copy(idx_hbm.at[pl.ds(off, GPW)], idx_v)
        pltpu.async_copy(tab_hbm.at[idx_v], rows_v, sem).wait()
        pltpu.sync_copy(rows_v, out_hbm.at[pl.ds(off, GPW)])


@functools.cache
def _sc_gather_call():
    return pl.kernel(
        _sc_gather_body,
        out_type=jax.ShapeDtypeStruct((GP, SW), jnp.float32),
        mesh=_vsmesh(),
        scratch_types=[
            pltpu.VMEM((GPW,), jnp.int32),
            pltpu.VMEM((GPW, SW), jnp.float32),
            pltpu.SemaphoreType.DMA,
        ],
    )


def _sc_gather(tab, idx):
    return _sc_gather_call()(tab, idx)


def _tc_prep_body(degp_ref, x_ref, w1_ref, h1_ref, ns_ref, nd_ref):
    degp = degp_ref[...]
    deg_o = 1.0 + degp[0, 0, :N] + degp[1, 0, :N]
    deg_i = 1.0 + degp[0, 1, :N] + degp[1, 1, :N]
    ns = lax.rsqrt(deg_o)
    nd = lax.rsqrt(deg_i)
    ns_ref[...] = ns
    nd_ref[...] = nd
    h1_ref[...] = jnp.dot(x_ref[...] * ns[:, None], w1_ref[...])


def _tc_prep(degp, x, w1):
    return pl.pallas_call(
        _tc_prep_body,
        out_shape=[
            jax.ShapeDtypeStruct((N, H), jnp.float32),
            jax.ShapeDtypeStruct((N,), jnp.float32),
            jax.ShapeDtypeStruct((N,), jnp.float32),
        ],
    )(degp, x, w1)


def _tc_layer_body(p_ref, h_ref, nd_ref, b_ref, wn_ref, ns_ref, rm_ref, g_ref, hn_ref, rmax_ref):
    p = p_ref[...]
    agg = p[0, :N] + p[1, :N] + h_ref[...]
    g = jnp.tanh(agg * nd_ref[...][:, None] + b_ref[...][None, :])
    g_ref[...] = g
    hn_ref[...] = jnp.dot(g * ns_ref[...][:, None], wn_ref[...])
    rmax_ref[...] = jnp.maximum(rm_ref[...], jnp.max(g, axis=1))


def _tc_layer(p, h, nd, b, wn, ns, rm):
    hn = wn.shape[1]
    return pl.pallas_call(
        _tc_layer_body,
        out_shape=[
            jax.ShapeDtypeStruct((N, H), jnp.float32),
            jax.ShapeDtypeStruct((N, hn), jnp.float32),
            jax.ShapeDtypeStruct((N,), jnp.float32),
        ],
    )(p, h, nd, b, wn, ns, rm)


def _tc_final_body(p_ref, h4_ref, nd_ref, b4_ref, rm_ref, g4_ref, rmax_ref):
    p = p_ref[...]
    agg = p[0, :N] + p[1, :N] + h4_ref[...]
    g4 = jnp.tanh(agg * nd_ref[...] + b4_ref[...])
    g4_ref[...] = g4
    rmax_ref[...] = jnp.maximum(rm_ref[...], g4)


def _tc_final(p4, h4, nd, b4, rm):
    return pl.pallas_call(
        _tc_final_body,
        out_shape=[
            jax.ShapeDtypeStruct((N,), jnp.float32),
            jax.ShapeDtypeStruct((N,), jnp.float32),
        ],
    )(p4, h4, nd, b4, rm)


def _partner(x, j, cols):
    if j < cols:
        lo = jnp.roll(x, -j, axis=1)
        hi = jnp.roll(x, j, axis=1)
    else:
        lo = jnp.roll(x, -(j // cols), axis=0)
        hi = jnp.roll(x, j // cols, axis=0)
    return lo, hi


def _tc_topk_body(keys_ref, idx_ref):
    R, C = TW // 128, 128
    kv = keys_ref[...]
    pos = lax.broadcasted_iota(jnp.int32, (R, C), 0) * C + lax.broadcasted_iota(
        jnp.int32, (R, C), 1
    )
    iv = pos
    kk = 2
    while kk <= TW:
        j = kk // 2
        while j >= 1:
            klo, khi = _partner(kv, j, C)
            ilo, ihi = _partner(iv, j, C)
            bit = (pos & j) == 0
            pk = jnp.where(bit, klo, khi)
            pi = jnp.where(bit, ilo, ihi)
            less = (kv > pk) | ((kv == pk) & (iv < pi))
            asc = (pos & kk) == 0
            keep = less == (bit == asc)
            kv = jnp.where(keep, kv, pk)
            iv = jnp.where(keep, iv, pi)
            j //= 2
        kk *= 2
    idx_ref[...] = iv


def _tc_topk(keys2d):
    return pl.pallas_call(
        _tc_topk_body,
        out_shape=jax.ShapeDtypeStruct((TW // 128, 128), jnp.int32),
    )(keys2d)


def _tc_head_body(rows_ref, wc1_ref, bc1_ref, apool_ref, wc2_ref, bc2_ref, out_ref):
    x = rows_ref[...]
    pos = lax.broadcasted_iota(jnp.int32, (K, SW), 1)
    kk = 2
    while kk <= SW:
        j = kk // 2
        while j >= 1:
            lo = jnp.roll(x, -j, axis=1)
            hi = jnp.roll(x, j, axis=1)
            bit = (pos & j) == 0
            px = jnp.where(bit, lo, hi)
            less = x < px
            asc = (pos & kk) == 0
            keep = less == (bit == asc)
            x = jnp.where(keep, x, px)
            j //= 2
        kk *= 2
    x = jnp.where(pos < TLD, x, 0.0)
    z = jnp.maximum(jnp.dot(x, wc1_ref[...]) + bc1_ref[...][None, :], 0.0)
    pooled = jnp.dot(apool_ref[...], z)
    t = wc2_ref[...] * pooled[None, :, :]
    out = jnp.maximum(jnp.sum(jnp.sum(t, axis=2), axis=1) + bc2_ref[...], 0.0)
    out_ref[...] = out[None, :]


def _tc_head(rows, wc1p, bc1, apool, wc2t, bc2):
    return pl.pallas_call(
        _tc_head_body,
        out_shape=jax.ShapeDtypeStruct((1, 32), jnp.float32),
    )(rows, wc1p, bc1, apool, wc2t, bc2)


def kernel(X, edge_index, W1, b1, W2, b2, W3, b3, W4, b4, Wc1, bc1, Wc2, bc2):
    padidx = (N + jnp.arange(EPAD - E, dtype=jnp.int32) % (NP - N)).astype(jnp.int32)
    src2 = jnp.concatenate([edge_index[0], padidx]).reshape(ECHUNKS, CH)
    dst2 = jnp.concatenate([edge_index[1], padidx]).reshape(ECHUNKS, CH)

    degp = _sc_degrees(src2, dst2)
    h1, ns, nd = _tc_prep(degp, X, W1)

    rowpad = ((0, NP - N), (0, 0))
    rm0 = jnp.full((N,), -jnp.inf, jnp.float32)
    p1 = _sc_rowscatter(jnp.pad(h1, rowpad), src2, dst2)
    g1, h2, rm1 = _tc_layer(p1, h1, nd, b1, W2, ns, rm0)
    p2 = _sc_rowscatter(jnp.pad(h2, rowpad), src2, dst2)
    g2, h3, rm2 = _tc_layer(p2, h2, nd, b2, W3, ns, rm1)
    p3 = _sc_rowscatter(jnp.pad(h3, rowpad), src2, dst2)
    g3, h4_2d, rm3 = _tc_layer(p3, h3, nd, b3, W4, ns, rm2)

    h4 = h4_2d.reshape(N)
    h4p = jnp.pad(h4, (0, NP - N))
    p4 = _sc_scalarscatter(h4p, src2, dst2)
    g4, rmax = _tc_final(p4, h4, nd, b4, rm3)

    keys2d = jnp.pad(rmax, (0, TW - N), constant_values=-jnp.inf).reshape(TW // 128, 128)
    idx2d = _tc_topk(keys2d)
    idx768 = jnp.pad(idx2d.reshape(-1)[:K], (0, GP - K))

    gcat = jnp.concatenate(
        [g1, g2, g3, g4[:, None], jnp.full((N, SW - TLD), PADV, jnp.float32)], axis=1
    )
    rows = _sc_gather(gcat, idx768)

    wc1p = jnp.pad(jnp.transpose(Wc1[:, 0, :]), ((0, SW - TLD), (0, 0)))
    apool = 0.5 * (
        (jnp.arange(K)[None, :] // 2) == jnp.arange(K // 2)[:, None]
    ).astype(jnp.float32)
    wc2t = jnp.transpose(Wc2, (0, 2, 1))

    return _tc_head(rows[:K], wc1p, bc1, apool, wc2t, bc2)


# speedup vs baseline: 15.2989x; 1.0114x over previous
import functools

import jax
import jax.numpy as jnp
from jax import lax
from jax.experimental import pallas as pl
from jax.experimental.pallas import tpu as pltpu
from jax.experimental.pallas import tpu_sc as plsc

N = 10000
NP = 10240
E = 320000
D_IN = 128
H = 128
K = 600
TLD = H * 3 + 1
NC, NS = 2, 16
NW = NC * NS
CH = 128
NCHUNK = E // CH
NROWCH = NP // CH
NZ = NROWCH // NS
GP = 768
GPW = 128
SW = 512
TW = 16384
PADV = 2.0


def _vsmesh():
    return plsc.VectorSubcoreMesh(
        core_axis_name="c", subcore_axis_name="s", num_cores=NC, num_subcores=NS
    )


def _fill_1d(ref, n, val):
    for j in range(n // 16):
        ref[pl.ds(j * 16, 16)] = jnp.full((16,), val, jnp.float32)


def _zero_rows(ref, nrow, ncol):
    def body(i, carry):
        for j in range(ncol // 16):
            ref[i, pl.ds(j * 16, 16)] = jnp.zeros((16,), jnp.float32)
        return carry

    lax.fori_loop(0, nrow, body, 0)


ECHUNKS = 2560
EPAD = ECHUNKS * CH
CPW = ECHUNKS // NW
SUB = 40
NB = CPW // SUB


def _sc_degrees_body(src2_hbm, dst2_hbm, out_hbm, idxs_v, idxd_v, ones_v, buf_v,
                     semg, sems, dego_sh, degi_sh):
    c = lax.axis_index("c")
    s = lax.axis_index("s")
    wid = s * NC + c
    _fill_1d(ones_v, CH, 1.0)
    _fill_1d(buf_v, CH, 0.0)

    def zbody(t, carry):
        off = pl.multiple_of((s * NZ + t) * CH, CH)
        pltpu.sync_copy(buf_v, dego_sh.at[pl.ds(off, CH)])
        pltpu.sync_copy(buf_v, degi_sh.at[pl.ds(off, CH)])
        return carry

    lax.fori_loop(0, NZ, zbody, 0)
    plsc.subcore_barrier()

    base = wid * CPW

    def batch(boff, nsub):
        pltpu.sync_copy(src2_hbm.at[pl.ds(boff, nsub)], idxs_v.at[pl.ds(0, nsub)])
        pltpu.sync_copy(dst2_hbm.at[pl.ds(boff, nsub)], idxd_v.at[pl.ds(0, nsub)])
        ds = [None] * nsub
        dd = [None] * nsub
        for j in range(nsub):
            if j >= 2:
                ds[j - 2].wait()
                dd[j - 2].wait()
            ds[j] = pltpu.async_copy(ones_v, dego_sh.at[idxs_v.at[j]], semg, add=True)
            dd[j] = pltpu.async_copy(ones_v, degi_sh.at[idxd_v.at[j]], sems, add=True)
        for j in range(max(0, nsub - 2), nsub):
            ds[j].wait()
            dd[j].wait()

    def bloop(b, carry):
        batch(base + b * SUB, SUB)
        return carry

    lax.fori_loop(0, NB, bloop, 0)
    plsc.subcore_barrier()

    def xbody(t, carry):
        off = pl.multiple_of((s * NZ + t) * CH, CH)
        pltpu.sync_copy(dego_sh.at[pl.ds(off, CH)], buf_v)
        pltpu.sync_copy(buf_v, out_hbm.at[c, 0, pl.ds(off, CH)])
        pltpu.sync_copy(degi_sh.at[pl.ds(off, CH)], buf_v)
        pltpu.sync_copy(buf_v, out_hbm.at[c, 1, pl.ds(off, CH)])
        return carry

    lax.fori_loop(0, NZ, xbody, 0)


@functools.cache
def _sc_degrees_call():
    return pl.kernel(
        _sc_degrees_body,
        out_type=jax.ShapeDtypeStruct((NC, 2, NP), jnp.float32),
        mesh=_vsmesh(),
        scratch_types=[
            pltpu.VMEM((SUB, CH), jnp.int32),
            pltpu.VMEM((SUB, CH), jnp.int32),
            pltpu.VMEM((CH,), jnp.float32),
            pltpu.VMEM((CH,), jnp.float32),
            pltpu.SemaphoreType.DMA,
            pltpu.SemaphoreType.DMA,
            pltpu.VMEM_SHARED((NP,), jnp.float32),
            pltpu.VMEM_SHARED((NP,), jnp.float32),
        ],
    )


def _sc_degrees(src2, dst2):
    return _sc_degrees_call()(src2, dst2)


def _sc_rowscatter_body(h_hbm, src2_hbm, dst2_hbm, out_hbm,
                        idxs_v, idxd_v, rows0_v, rows1_v,
                        semg0, semg1, sems0, sems1, agg_sh):
    c = lax.axis_index("c")
    s = lax.axis_index("s")
    wid = s * NC + c
    _zero_rows(rows0_v, CH, H)

    def zbody(t, carry):
        off = pl.multiple_of((s * NZ + t) * CH, CH)
        pltpu.sync_copy(rows0_v, agg_sh.at[pl.ds(off, CH)])
        return carry

    lax.fori_loop(0, NZ, zbody, 0)
    plsc.subcore_barrier()

    base = wid * CPW
    rbufs = (rows0_v, rows1_v)
    gsems = (semg0, semg1)
    ssems = (sems0, sems1)

    def batch(boff, nsub):
        pltpu.sync_copy(src2_hbm.at[pl.ds(boff, nsub)], idxs_v.at[pl.ds(0, nsub)])
        pltpu.sync_copy(dst2_hbm.at[pl.ds(boff, nsub)], idxd_v.at[pl.ds(0, nsub)])
        gd = [None] * nsub
        sd = [None] * nsub
        gd[0] = pltpu.async_copy(h_hbm.at[idxs_v.at[0]], rbufs[0], gsems[0])
        for j in range(nsub):
            gd[j].wait()
            if j > 0:
                sd[j - 1].wait()
            if j + 1 < nsub:
                gd[j + 1] = pltpu.async_copy(
                    h_hbm.at[idxs_v.at[j + 1]], rbufs[(j + 1) % 2], gsems[(j + 1) % 2]
                )
            sd[j] = pltpu.async_copy(
                rbufs[j % 2], agg_sh.at[idxd_v.at[j]], ssems[j % 2], add=True
            )
        sd[nsub - 1].wait()

    def bloop(b, carry):
        batch(base + b * SUB, SUB)
        return carry

    lax.fori_loop(0, NB, bloop, 0)
    plsc.subcore_barrier()

    def xbody(t, carry):
        off = pl.multiple_of((s * NZ + t) * CH, CH)
        pltpu.sync_copy(agg_sh.at[pl.ds(off, CH)], rows0_v)
        pltpu.sync_copy(rows0_v, out_hbm.at[c, pl.ds(off, CH)])
        return carry

    lax.fori_loop(0, NZ, xbody, 0)


@functools.cache
def _sc_rowscatter_call():
    return pl.kernel(
        _sc_rowscatter_body,
        out_type=jax.ShapeDtypeStruct((NC, NP, H), jnp.float32),
        mesh=_vsmesh(),
        scratch_types=[
            pltpu.VMEM((SUB, CH), jnp.int32),
            pltpu.VMEM((SUB, CH), jnp.int32),
            pltpu.VMEM((CH, H), jnp.float32),
            pltpu.VMEM((CH, H), jnp.float32),
            pltpu.SemaphoreType.DMA,
            pltpu.SemaphoreType.DMA,
            pltpu.SemaphoreType.DMA,
            pltpu.SemaphoreType.DMA,
            pltpu.VMEM_SHARED((NP, H), jnp.float32),
        ],
    )


def _sc_rowscatter(h, src2, dst2):
    return _sc_rowscatter_call()(h, src2, dst2)


def _sc_scalarscatter_body(h4_hbm, src2_hbm, dst2_hbm, out_hbm,
                           idxs_v, idxd_v, vals0_v, vals1_v, buf_v,
                           semg0, semg1, sems0, sems1, acc_sh):
    c = lax.axis_index("c")
    s = lax.axis_index("s")
    wid = s * NC + c
    _fill_1d(buf_v, CH, 0.0)

    def zbody(t, carry):
        off = pl.multiple_of((s * NZ + t) * CH, CH)
        pltpu.sync_copy(buf_v, acc_sh.at[pl.ds(off, CH)])
        return carry

    lax.fori_loop(0, NZ, zbody, 0)
    plsc.subcore_barrier()

    base = wid * CPW
    vbufs = (vals0_v, vals1_v)
    gsems = (semg0, semg1)
    ssems = (sems0, sems1)

    def batch(boff, nsub):
        pltpu.sync_copy(src2_hbm.at[pl.ds(boff, nsub)], idxs_v.at[pl.ds(0, nsub)])
        pltpu.sync_copy(dst2_hbm.at[pl.ds(boff, nsub)], idxd_v.at[pl.ds(0, nsub)])
        gd = [None] * nsub
        sd = [None] * nsub
        gd[0] = pltpu.async_copy(h4_hbm.at[idxs_v.at[0]], vbufs[0], gsems[0])
        for j in range(nsub):
            gd[j].wait()
            if j > 0:
                sd[j - 1].wait()
            if j + 1 < nsub:
                gd[j + 1] = pltpu.async_copy(
                    h4_hbm.at[idxs_v.at[j + 1]], vbufs[(j + 1) % 2], gsems[(j + 1) % 2]
                )
            sd[j] = pltpu.async_copy(
                vbufs[j % 2], acc_sh.at[idxd_v.at[j]], ssems[j % 2], add=True
            )
        sd[nsub - 1].wait()

    def bloop(b, carry):
        batch(base + b * SUB, SUB)
        return carry

    lax.fori_loop(0, NB, bloop, 0)
    plsc.subcore_barrier()

    def xbody(t, carry):
        off = pl.multiple_of((s * NZ + t) * CH, CH)
        pltpu.sync_copy(acc_sh.at[pl.ds(off, CH)], buf_v)
        pltpu.sync_copy(buf_v, out_hbm.at[c, pl.ds(off, CH)])
        return carry

    lax.fori_loop(0, NZ, xbody, 0)


@functools.cache
def _sc_scalarscatter_call():
    return pl.kernel(
        _sc_scalarscatter_body,
        out_type=jax.ShapeDtypeStruct((NC, NP), jnp.float32),
        mesh=_vsmesh(),
        scratch_types=[
            pltpu.VMEM((SUB, CH), jnp.int32),
            pltpu.VMEM((SUB, CH), jnp.int32),
            pltpu.VMEM((CH,), jnp.float32),
            pltpu.VMEM((CH,), jnp.float32),
            pltpu.VMEM((CH,), jnp.float32),
            pltpu.SemaphoreType.DMA,
            pltpu.SemaphoreType.DMA,
            pltpu.SemaphoreType.DMA,
            pltpu.SemaphoreType.DMA,
            pltpu.VMEM_SHARED((NP,), jnp.float32),
        ],
    )


def _sc_scalarscatter(h4, src2, dst2):
    return _sc_scalarscatter_call()(h4, src2, dst2)


def _sc_gather_body(tab_hbm, idx_hbm, out_hbm, idx_v, rows_v, sem):
    c = lax.axis_index("c")
    s = lax.axis_index("s")
    wid = s * NC + c

    @pl.when(wid < GP // GPW)
    def _():
        off = pl.multiple_of(wid * GPW, CH)
        pltpu.sync_copy(idx_hbm.at[pl.ds(off, GPW)], idx_v)
        pltpu.async_copy(tab_hbm.at[idx_v], rows_v, sem).wait()
        pltpu.sync_copy(rows_v, out_hbm.at[pl.ds(off, GPW)])


@functools.cache
def _sc_gather_call():
    return pl.kernel(
        _sc_gather_body,
        out_type=jax.ShapeDtypeStruct((GP, SW), jnp.float32),
        mesh=_vsmesh(),
        scratch_types=[
            pltpu.VMEM((GPW,), jnp.int32),
            pltpu.VMEM((GPW, SW), jnp.float32),
            pltpu.SemaphoreType.DMA,
        ],
    )


def _sc_gather(tab, idx):
    return _sc_gather_call()(tab, idx)


def _tc_prep_body(degp_ref, x_ref, w1_ref, h1_ref, ns_ref, nd_ref):
    degp = degp_ref[...]
    deg_o = 1.0 + degp[0, 0, :N] + degp[1, 0, :N]
    deg_i = 1.0 + degp[0, 1, :N] + degp[1, 1, :N]
    ns = lax.rsqrt(deg_o)
    nd = lax.rsqrt(deg_i)
    ns_ref[...] = ns
    nd_ref[...] = nd
    h1_ref[pl.ds(0, N)] = jnp.dot(x_ref[...] * ns[:, None], w1_ref[...])
    h1_ref[pl.ds(N, NP - N)] = jnp.zeros((NP - N, H), jnp.float32)


def _tc_prep(degp, x, w1):
    return pl.pallas_call(
        _tc_prep_body,
        out_shape=[
            jax.ShapeDtypeStruct((NP, H), jnp.float32),
            jax.ShapeDtypeStruct((N,), jnp.float32),
            jax.ShapeDtypeStruct((N,), jnp.float32),
        ],
    )(degp, x, w1)


def _tc_layer_body(p_ref, h_ref, nd_ref, b_ref, wn_ref, ns_ref, rm_ref, g_ref, hn_ref, rmax_ref):
    p = p_ref[...]
    agg = p[0, :N] + p[1, :N] + h_ref[pl.ds(0, N)]
    g = jnp.tanh(agg * nd_ref[...][:, None] + b_ref[...][None, :])
    g_ref[...] = g
    hn = jnp.dot(g * ns_ref[...][:, None], wn_ref[...])
    if hn_ref.shape[0] == NP:
        hn_ref[pl.ds(0, N)] = hn
        hn_ref[pl.ds(N, NP - N)] = jnp.zeros((NP - N, hn.shape[1]), jnp.float32)
    else:
        hn_ref[...] = hn
    rmax_ref[...] = jnp.maximum(rm_ref[...], jnp.max(g, axis=1))


def _tc_layer(p, h, nd, b, wn, ns, rm, pad_out=True):
    hn = wn.shape[1]
    return pl.pallas_call(
        _tc_layer_body,
        out_shape=[
            jax.ShapeDtypeStruct((N, H), jnp.float32),
            jax.ShapeDtypeStruct((NP if pad_out else N, hn), jnp.float32),
            jax.ShapeDtypeStruct((N,), jnp.float32),
        ],
    )(p, h, nd, b, wn, ns, rm)


def _tc_final_body(p_ref, h4_ref, nd_ref, b4_ref, rm_ref, g4_ref, rmax_ref):
    p = p_ref[...]
    agg = p[0, :N] + p[1, :N] + h4_ref[...]
    g4 = jnp.tanh(agg * nd_ref[...] + b4_ref[...])
    g4_ref[...] = g4
    rmax_ref[...] = jnp.maximum(rm_ref[...], g4)


def _tc_final(p4, h4, nd, b4, rm):
    return pl.pallas_call(
        _tc_final_body,
        out_shape=[
            jax.ShapeDtypeStruct((N,), jnp.float32),
            jax.ShapeDtypeStruct((N,), jnp.float32),
        ],
    )(p4, h4, nd, b4, rm)


def _partner(x, j, cols):
    if j < cols:
        lo = jnp.roll(x, -j, axis=1)
        hi = jnp.roll(x, j, axis=1)
    else:
        lo = jnp.roll(x, -(j // cols), axis=0)
        hi = jnp.roll(x, j // cols, axis=0)
    return lo, hi


def _tc_topk_body(keys_ref, idx_ref):
    R, C = TW // 128, 128
    kv = keys_ref[...]
    pos = lax.broadcasted_iota(jnp.int32, (R, C), 0) * C + lax.broadcasted_iota(
        jnp.int32, (R, C), 1
    )
    iv = pos
    kk = 2
    while kk <= TW:
        j = kk // 2
        while j >= 1:
            klo, khi = _partner(kv, j, C)
            ilo, ihi = _partner(iv, j, C)
            bit = (pos & j) == 0
            pk = jnp.where(bit, klo, khi)
            pi = jnp.where(bit, ilo, ihi)
            less = (kv > pk) | ((kv == pk) & (iv < pi))
            asc = (pos & kk) == 0
            keep = less == (bit == asc)
            kv = jnp.where(keep, kv, pk)
            iv = jnp.where(keep, iv, pi)
            j //= 2
        kk *= 2
    idx_ref[...] = iv


def _tc_topk(keys2d):
    return pl.pallas_call(
        _tc_topk_body,
        out_shape=jax.ShapeDtypeStruct((TW // 128, 128), jnp.int32),
    )(keys2d)


def _tc_head_body(rows_ref, wc1_ref, bc1_ref, apool_ref, wc2_ref, bc2_ref, out_ref):
    x = rows_ref[...]
    pos = lax.broadcasted_iota(jnp.int32, (K, SW), 1)
    kk = 2
    while kk <= SW:
        j = kk // 2
        while j >= 1:
            lo = jnp.roll(x, -j, axis=1)
            hi = jnp.roll(x, j, axis=1)
            bit = (pos & j) == 0
            px = jnp.where(bit, lo, hi)
            less = x < px
            asc = (pos & kk) == 0
            keep = less == (bit == asc)
            x = jnp.where(keep, x, px)
            j //= 2
        kk *= 2
    x = jnp.where(pos < TLD, x, 0.0)
    z = jnp.maximum(jnp.dot(x, wc1_ref[...]) + bc1_ref[...][None, :], 0.0)
    pooled = jnp.dot(apool_ref[...], z)
    t = wc2_ref[...] * pooled[None, :, :]
    out = jnp.maximum(jnp.sum(jnp.sum(t, axis=2), axis=1) + bc2_ref[...], 0.0)
    out_ref[...] = out[None, :]


def _tc_head(rows, wc1p, bc1, apool, wc2t, bc2):
    return pl.pallas_call(
        _tc_head_body,
        out_shape=jax.ShapeDtypeStruct((1, 32), jnp.float32),
    )(rows, wc1p, bc1, apool, wc2t, bc2)


def kernel(X, edge_index, W1, b1, W2, b2, W3, b3, W4, b4, Wc1, bc1, Wc2, bc2):
    padidx = (N + jnp.arange(EPAD - E, dtype=jnp.int32) % (NP - N)).astype(jnp.int32)
    src2 = jnp.concatenate([edge_index[0], padidx]).reshape(ECHUNKS, CH)
    dst2 = jnp.concatenate([edge_index[1], padidx]).reshape(ECHUNKS, CH)

    degp = _sc_degrees(src2, dst2)
    h1, ns, nd = _tc_prep(degp, X, W1)

    rm0 = jnp.full((N,), -jnp.inf, jnp.float32)
    p1 = _sc_rowscatter(h1, src2, dst2)
    g1, h2, rm1 = _tc_layer(p1, h1, nd, b1, W2, ns, rm0)
    p2 = _sc_rowscatter(h2, src2, dst2)
    g2, h3, rm2 = _tc_layer(p2, h2, nd, b2, W3, ns, rm1)
    p3 = _sc_rowscatter(h3, src2, dst2)
    g3, h4p_2d, rm3 = _tc_layer(p3, h3, nd, b3, W4, ns, rm2)

    h4p = h4p_2d.reshape(NP)
    h4 = h4p[:N]
    p4 = _sc_scalarscatter(h4p, src2, dst2)
    g4, rmax = _tc_final(p4, h4, nd, b4, rm3)

    keys2d = jnp.pad(rmax, (0, TW - N), constant_values=-jnp.inf).reshape(TW // 128, 128)
    idx2d = _tc_topk(keys2d)
    idx768 = jnp.pad(idx2d.reshape(-1)[:K], (0, GP - K))

    gcat = jnp.concatenate(
        [g1, g2, g3, g4[:, None], jnp.full((N, SW - TLD), PADV, jnp.float32)], axis=1
    )
    rows = _sc_gather(gcat, idx768)

    wc1p = jnp.pad(jnp.transpose(Wc1[:, 0, :]), ((0, SW - TLD), (0, 0)))
    apool = 0.5 * (
        (jnp.arange(K)[None, :] // 2) == jnp.arange(K // 2)[:, None]
    ).astype(jnp.float32)
    wc2t = jnp.transpose(Wc2, (0, 2, 1))

    return _tc_head(rows[:K], wc1p, bc1, apool, wc2t, bc2)
